# Initial kernel scaffold; baseline (speedup 1.0000x reference)
import jax
import jax.numpy as jnp
from jax.experimental import pallas as pl


def kernel(x, edge_index, edge_attr, train_masked_y, eval_masked_y, params):
    raise NotImplementedError("write your pallas kernel here")


# speedup vs baseline: 4.8382x; 4.8382x over previous
import functools
import math

import jax
import jax.numpy as jnp
from jax import lax
from jax.experimental import pallas as pl
from jax.experimental.pallas import tpu as pltpu
from jax.experimental.pallas import tpu_sc as plsc

N = 10000
E = 160000
IN_DIM = 8
LABEL_DIM = 112

NPAD = 10240
DUMP = N
EPAD = 163840
NC, NS = 2, 16
NW = NC * NS
EPW = EPAD // NW
CHUNK = 128
NCHUNKS = EPW // CHUNK
ROWS_PER_TILE = NPAD // NS
NODE_BLK = 1024
EDGE_BLK = 2048
AW = 128

_f32 = jnp.float32


def _proj_body(h_ref, y_ref, wq, bq, wkn, bkn, wv, bv, wl, bl, ws, bs,
               wket, bkec, dtab_ref, stab_ref, skip_ref, *, C, SW):
    h = h_ref[...]
    y = y_ref[...]
    b = h.shape[0]
    q = jnp.dot(h, wq[...], preferred_element_type=_f32) + bq[...]
    kn = jnp.dot(h, wkn[...], preferred_element_type=_f32) + bkn[...]
    v = jnp.dot(h, wv[...], preferred_element_type=_f32) + bv[...]
    labp = jnp.dot(y, wl[...], preferred_element_type=_f32) + bl[...]
    w = v + labp
    skip_ref[...] = jnp.dot(h, ws[...], preferred_element_type=_f32) + bs[...]
    u = jnp.dot(q, wket[...], preferred_element_type=_f32)
    cterm = jnp.dot(q, bkec[...], preferred_element_type=_f32)
    dparts = [q, u, cterm]
    if 128 - C - 9 > 0:
        dparts.append(jnp.zeros((b, 128 - C - 9), _f32))
    dtab_ref[...] = jnp.concatenate(dparts, axis=1)
    sparts = [kn, w]
    if SW - 2 * C > 0:
        sparts.append(jnp.zeros((b, SW - 2 * C), _f32))
    stab_ref[...] = jnp.concatenate(sparts, axis=1)


def _proj(h, y, p, C, SW):
    din = h.shape[1]
    grid = NPAD // NODE_BLK
    row_spec = lambda width: pl.BlockSpec((NODE_BLK, width), lambda i: (i, 0))
    full = lambda a: pl.BlockSpec(a.shape, lambda i: (0,) * a.ndim)
    wq, bq = p["Wq"], p["bq"].reshape(1, C)
    wkn, bkn = p["Wkn"], p["bkn"].reshape(1, C)
    wv, bv = p["Wv"], p["bv"].reshape(1, C)
    wl, bl = p["Wl"], p["bl"].reshape(1, C)
    ws, bs = p["Ws"], p["bs"].reshape(1, C)
    wket = p["Wke"].T
    bkec = p["bke"].reshape(C, 1)
    args = (h, y, wq, bq, wkn, bkn, wv, bv, wl, bl, ws, bs, wket, bkec)
    in_specs = [row_spec(din), row_spec(LABEL_DIM)] + [full(a) for a in args[2:]]
    out_sd = [jax.ShapeDtypeStruct((NPAD, 128), _f32),
              jax.ShapeDtypeStruct((NPAD, SW), _f32),
              jax.ShapeDtypeStruct((NPAD, C), _f32)]
    out_specs = [row_spec(128), row_spec(SW), row_spec(C)]
    return pl.pallas_call(
        functools.partial(_proj_body, C=C, SW=SW),
        grid=(grid,), in_specs=in_specs, out_specs=out_specs,
        out_shape=out_sd)(*args)


def _logits_body(dg, sg, ea, l_ref, m_ref, mmax, *, C, inv_sqrt_c):
    i = pl.program_id(0)
    d = dg[...]
    qg = d[:, :C]
    ubg = d[:, C:C + 16]
    kng = sg[...][:, :C]
    dots = (jnp.sum(qg * kng, axis=1, keepdims=True)
            + jnp.sum(ubg * ea[...], axis=1, keepdims=True))
    l = dots * inv_sqrt_c
    l_ref[...] = l
    bm = jnp.max(l)

    @pl.when(i == 0)
    def _():
        mmax[0] = bm

    @pl.when(i > 0)
    def _():
        mmax[0] = jnp.maximum(mmax[0], bm)

    @pl.when(i == pl.num_programs(0) - 1)
    def _():
        m_ref[...] = jnp.broadcast_to(mmax[0], (1, 1))


def _logits(dg, sg, ea, C, SW):
    grid = EPAD // EDGE_BLK
    spec = lambda width: pl.BlockSpec((EDGE_BLK, width), lambda i: (i, 0))
    return pl.pallas_call(
        functools.partial(_logits_body, C=C, inv_sqrt_c=1.0 / math.sqrt(C)),
        grid=(grid,),
        in_specs=[spec(128), spec(SW), spec(16)],
        out_specs=[spec(1), pl.BlockSpec((1, 1), lambda i: (0, 0))],
        out_shape=[jax.ShapeDtypeStruct((EPAD, 1), _f32),
                   jax.ShapeDtypeStruct((1, 1), _f32)],
        scratch_shapes=[pltpu.SMEM((1,), _f32)])(dg, sg, ea)


def _msg_body(l_ref, m_ref, sg_ref, msg_ref, *, C):
    p = jnp.exp(l_ref[...] - m_ref[0, 0])
    wg = sg_ref[...][:, C:2 * C]
    msg_ref[...] = jnp.concatenate(
        [wg * p, p, jnp.zeros((p.shape[0], AW - C - 1), _f32)], axis=1)


def _msg(l, m, sg, C, SW):
    grid = EPAD // EDGE_BLK
    spec = lambda width: pl.BlockSpec((EDGE_BLK, width), lambda i: (i, 0))
    return pl.pallas_call(
        functools.partial(_msg_body, C=C), grid=(grid,),
        in_specs=[spec(1), pl.BlockSpec((1, 1), lambda i: (0, 0)), spec(SW)],
        out_specs=spec(AW),
        out_shape=jax.ShapeDtypeStruct((EPAD, AW), _f32))(l, m, sg)


def _combine_body(a0_ref, a1_ref, skip_ref, out_ref, *, C, relu):
    a = a0_ref[...] + a1_ref[...]
    o = a[:, :C] / (a[:, C:C + 1] + 1e-16) + skip_ref[...]
    if relu:
        o = jnp.maximum(o, 0.0)
    out_ref[...] = o


def _combine(acc0, acc1, skip, C, relu):
    grid = NPAD // NODE_BLK
    spec = lambda width: pl.BlockSpec((NODE_BLK, width), lambda i: (i, 0))
    return pl.pallas_call(
        functools.partial(_combine_body, C=C, relu=relu), grid=(grid,),
        in_specs=[spec(AW), spec(AW), spec(C)],
        out_specs=spec(C),
        out_shape=jax.ShapeDtypeStruct((NPAD, C), _f32))(acc0, acc1, skip)


def _gather_body(dtab_hbm, stab_hbm, dst2, src2, dg_hbm, sg_hbm,
                 dstb, srcb, drows, srows, sem, *, SW):
    c = lax.axis_index("c")
    s = lax.axis_index("s")
    wid = s * NC + c
    pltpu.sync_copy(dst2.at[pl.ds(wid * NCHUNKS, NCHUNKS)], dstb)
    pltpu.sync_copy(src2.at[pl.ds(wid * NCHUNKS, NCHUNKS)], srcb)

    def chunk(j, carry):
        ebase = wid * EPW + j * CHUNK
        cp1 = pltpu.async_copy(dtab_hbm.at[dstb.at[j]], drows, sem)
        cp2 = pltpu.async_copy(stab_hbm.at[srcb.at[j]], srows, sem)
        cp1.wait()
        cp2.wait()
        pltpu.sync_copy(drows, dg_hbm.at[pl.ds(ebase, CHUNK)])
        pltpu.sync_copy(srows, sg_hbm.at[pl.ds(ebase, CHUNK)])
        return carry

    lax.fori_loop(0, NCHUNKS, chunk, 0)


def _gather(dtab, stab, dst2, src2, SW):
    mesh = plsc.VectorSubcoreMesh(core_axis_name="c", subcore_axis_name="s")
    kfn = pl.kernel(
        functools.partial(_gather_body, SW=SW),
        out_type=[jax.ShapeDtypeStruct((EPAD, 128), _f32),
                  jax.ShapeDtypeStruct((EPAD, SW), _f32)],
        mesh=mesh,
        scratch_types=[
            pltpu.VMEM((NCHUNKS, CHUNK), jnp.int32),
            pltpu.VMEM((NCHUNKS, CHUNK), jnp.int32),
            pltpu.VMEM((CHUNK, 128), _f32),
            pltpu.VMEM((CHUNK, SW), _f32),
            pltpu.SemaphoreType.DMA,
        ])
    return kfn(dtab, stab, dst2, src2)


def _scatter_body(msg_hbm, dst2, zrow_hbm, accs_hbm, dstb, msgv, zv, acc_sh):
    c = lax.axis_index("c")
    s = lax.axis_index("s")
    wid = s * NC + c
    pltpu.sync_copy(dst2.at[pl.ds(wid * NCHUNKS, NCHUNKS)], dstb)
    pltpu.sync_copy(zrow_hbm, zv)
    for r in range(ROWS_PER_TILE // CHUNK):
        pltpu.sync_copy(zv, acc_sh.at[pl.ds(s * ROWS_PER_TILE + r * CHUNK, CHUNK)])
    plsc.subcore_barrier()

    def chunk(j, carry):
        ebase = wid * EPW + j * CHUNK
        pltpu.sync_copy(msg_hbm.at[pl.ds(ebase, CHUNK)], msgv)
        pltpu.sync_copy(msgv, acc_sh.at[dstb.at[j]], add=True)
        return carry

    lax.fori_loop(0, NCHUNKS, chunk, 0)
    plsc.subcore_barrier()
    for r in range(ROWS_PER_TILE // CHUNK):
        rows = s * ROWS_PER_TILE + r * CHUNK
        pltpu.sync_copy(acc_sh.at[pl.ds(rows, CHUNK)], msgv)
        pltpu.sync_copy(msgv, accs_hbm.at[c, pl.ds(rows, CHUNK)])


def _scatter(msg, dst2):
    mesh = plsc.VectorSubcoreMesh(core_axis_name="c", subcore_axis_name="s")
    zrow = jnp.zeros((CHUNK, AW), _f32)
    kfn = pl.kernel(
        _scatter_body,
        out_type=jax.ShapeDtypeStruct((NC, NPAD, AW), _f32),
        mesh=mesh,
        scratch_types=[
            pltpu.VMEM((NCHUNKS, CHUNK), jnp.int32),
            pltpu.VMEM((CHUNK, AW), _f32),
            pltpu.VMEM((CHUNK, AW), _f32),
            pltpu.VMEM_SHARED((NPAD, AW), _f32),
        ])
    return kfn(msg, dst2, zrow)


def kernel(x, edge_index, edge_attr, train_masked_y, eval_masked_y, params):
    del train_masked_y
    src = edge_index[0]
    dst = edge_index[1]
    h = jnp.zeros((NPAD, IN_DIM), _f32).at[:N].set(x)
    y = jnp.zeros((NPAD, LABEL_DIM), _f32).at[:N].set(eval_masked_y)
    ea = jnp.concatenate(
        [edge_attr, jnp.ones((E, 1), _f32), jnp.zeros((E, 7), _f32)], axis=1)
    ea = jnp.zeros((EPAD, 16), _f32).at[:E].set(ea)
    src2 = jnp.zeros((EPAD,), jnp.int32).at[:E].set(src).reshape(EPAD // CHUNK, CHUNK)
    dst2 = jnp.full((EPAD,), DUMP, jnp.int32).at[:E].set(dst).reshape(EPAD // CHUNK, CHUNK)

    layers = params["layers"]
    for i, (C, relu) in enumerate([(64, True), (64, True), (112, False)]):
        SW = 128 if 2 * C <= 128 else 256
        p = layers[i]
        dtab, stab, skip = _proj(h, y, p, C, SW)
        dg, sg = _gather(dtab, stab, dst2, src2, SW)
        l, m = _logits(dg, sg, ea, C, SW)
        msg = _msg(l, m, sg, C, SW)
        accs = _scatter(msg, dst2)
        h = _combine(accs[0], accs[1], skip, C, relu)
    return h[:N]


# speedup vs baseline: 5.7738x; 1.1934x over previous
import functools
import math

import jax
import jax.numpy as jnp
from jax import lax
from jax.experimental import pallas as pl
from jax.experimental.pallas import tpu as pltpu
from jax.experimental.pallas import tpu_sc as plsc

N = 10000
E = 160000
IN_DIM = 8
LABEL_DIM = 112

NPAD = 10240
DUMP = N
EPAD = 163840
NC, NS = 2, 16
NW = NC * NS
EPW = EPAD // NW
CHUNK = 128
NCHUNKS = EPW // CHUNK
ROWS_PER_TILE = NPAD // NS
NODE_BLK = 1024
EDGE_BLK = 2048
AW = 128

_f32 = jnp.float32


def _proj_body(h_ref, y_ref, wq, bq, wkn, bkn, wv, bv, wl, bl, ws, bs,
               wket, bkec, dtab_ref, stab_ref, skip_ref, *, C, SW):
    h = h_ref[...]
    y = y_ref[...]
    b = h.shape[0]
    q = jnp.dot(h, wq[...], preferred_element_type=_f32) + bq[...]
    kn = jnp.dot(h, wkn[...], preferred_element_type=_f32) + bkn[...]
    v = jnp.dot(h, wv[...], preferred_element_type=_f32) + bv[...]
    labp = jnp.dot(y, wl[...], preferred_element_type=_f32) + bl[...]
    w = v + labp
    skip_ref[...] = jnp.dot(h, ws[...], preferred_element_type=_f32) + bs[...]
    u = jnp.dot(q, wket[...], preferred_element_type=_f32)
    cterm = jnp.dot(q, bkec[...], preferred_element_type=_f32)
    dparts = [q, u, cterm]
    if 128 - C - 9 > 0:
        dparts.append(jnp.zeros((b, 128 - C - 9), _f32))
    dtab_ref[...] = jnp.concatenate(dparts, axis=1)
    sparts = [kn, w]
    if SW - 2 * C > 0:
        sparts.append(jnp.zeros((b, SW - 2 * C), _f32))
    stab_ref[...] = jnp.concatenate(sparts, axis=1)


def _proj(h, y, p, C, SW):
    din = h.shape[1]
    grid = NPAD // NODE_BLK
    row_spec = lambda width: pl.BlockSpec((NODE_BLK, width), lambda i: (i, 0))
    full = lambda a: pl.BlockSpec(a.shape, lambda i: (0,) * a.ndim)
    wq, bq = p["Wq"], p["bq"].reshape(1, C)
    wkn, bkn = p["Wkn"], p["bkn"].reshape(1, C)
    wv, bv = p["Wv"], p["bv"].reshape(1, C)
    wl, bl = p["Wl"], p["bl"].reshape(1, C)
    ws, bs = p["Ws"], p["bs"].reshape(1, C)
    wket = p["Wke"].T
    bkec = p["bke"].reshape(C, 1)
    args = (h, y, wq, bq, wkn, bkn, wv, bv, wl, bl, ws, bs, wket, bkec)
    in_specs = [row_spec(din), row_spec(LABEL_DIM)] + [full(a) for a in args[2:]]
    out_sd = [jax.ShapeDtypeStruct((NPAD, 128), _f32),
              jax.ShapeDtypeStruct((NPAD, SW), _f32),
              jax.ShapeDtypeStruct((NPAD, C), _f32)]
    out_specs = [row_spec(128), row_spec(SW), row_spec(C)]
    return pl.pallas_call(
        functools.partial(_proj_body, C=C, SW=SW),
        grid=(grid,), in_specs=in_specs, out_specs=out_specs,
        out_shape=out_sd)(*args)


def _logits_body(dg, sg, ea, l_ref, m_ref, mmax, *, C, inv_sqrt_c):
    i = pl.program_id(0)
    d = dg[...]
    qg = d[:, :C]
    ubg = d[:, C:C + 16]
    kng = sg[...][:, :C]
    dots = (jnp.sum(qg * kng, axis=1, keepdims=True)
            + jnp.sum(ubg * ea[...], axis=1, keepdims=True))
    l = dots * inv_sqrt_c
    l_ref[...] = l
    bm = jnp.max(l)

    @pl.when(i == 0)
    def _():
        mmax[0] = bm

    @pl.when(i > 0)
    def _():
        mmax[0] = jnp.maximum(mmax[0], bm)

    @pl.when(i == pl.num_programs(0) - 1)
    def _():
        m_ref[...] = jnp.broadcast_to(mmax[0], (1, 1))


def _logits(dg, sg, ea, C, SW):
    grid = EPAD // EDGE_BLK
    spec = lambda width: pl.BlockSpec((EDGE_BLK, width), lambda i: (i, 0))
    return pl.pallas_call(
        functools.partial(_logits_body, C=C, inv_sqrt_c=1.0 / math.sqrt(C)),
        grid=(grid,),
        in_specs=[spec(128), spec(SW), spec(16)],
        out_specs=[spec(1), pl.BlockSpec((1, 1), lambda i: (0, 0))],
        out_shape=[jax.ShapeDtypeStruct((EPAD, 1), _f32),
                   jax.ShapeDtypeStruct((1, 1), _f32)],
        scratch_shapes=[pltpu.SMEM((1,), _f32)])(dg, sg, ea)


def _msg_body(l_ref, m_ref, sg_ref, msg_ref, *, C):
    p = jnp.exp(l_ref[...] - m_ref[0, 0])
    wg = sg_ref[...][:, C:2 * C]
    msg_ref[...] = jnp.concatenate(
        [wg * p, p, jnp.zeros((p.shape[0], AW - C - 1), _f32)], axis=1)


def _msg(l, m, sg, C, SW):
    grid = EPAD // EDGE_BLK
    spec = lambda width: pl.BlockSpec((EDGE_BLK, width), lambda i: (i, 0))
    return pl.pallas_call(
        functools.partial(_msg_body, C=C), grid=(grid,),
        in_specs=[spec(1), pl.BlockSpec((1, 1), lambda i: (0, 0)), spec(SW)],
        out_specs=spec(AW),
        out_shape=jax.ShapeDtypeStruct((EPAD, AW), _f32))(l, m, sg)


def _combine_body(a0_ref, a1_ref, skip_ref, out_ref, *, C, relu):
    a = a0_ref[...] + a1_ref[...]
    o = a[:, :C] / (a[:, C:C + 1] + 1e-16) + skip_ref[...]
    if relu:
        o = jnp.maximum(o, 0.0)
    out_ref[...] = o


def _combine(acc0, acc1, skip, C, relu):
    grid = NPAD // NODE_BLK
    spec = lambda width: pl.BlockSpec((NODE_BLK, width), lambda i: (i, 0))
    return pl.pallas_call(
        functools.partial(_combine_body, C=C, relu=relu), grid=(grid,),
        in_specs=[spec(AW), spec(AW), spec(C)],
        out_specs=spec(C),
        out_shape=jax.ShapeDtypeStruct((NPAD, C), _f32))(acc0, acc1, skip)


GNBUF = 2


def _gather_body(dtab_hbm, stab_hbm, dst2, src2, dg_hbm, sg_hbm, *scr, SW):
    dstb, srcb = scr[0], scr[1]
    dbuf = scr[2:2 + GNBUF]
    sbuf = scr[2 + GNBUF:2 + 2 * GNBUF]
    gsd = scr[2 + 2 * GNBUF:2 + 3 * GNBUF]
    gss = scr[2 + 3 * GNBUF:2 + 4 * GNBUF]
    wsd = scr[2 + 4 * GNBUF:2 + 5 * GNBUF]
    wss = scr[2 + 5 * GNBUF:2 + 6 * GNBUF]
    c = lax.axis_index("c")
    s = lax.axis_index("s")
    wid = s * NC + c
    pltpu.sync_copy(dst2.at[pl.ds(wid * NCHUNKS, NCHUNKS)], dstb)
    pltpu.sync_copy(src2.at[pl.ds(wid * NCHUNKS, NCHUNKS)], srcb)

    for b in range(GNBUF):
        pltpu.async_copy(dtab_hbm.at[dstb.at[b]], dbuf[b], gsd[b])
        pltpu.async_copy(stab_hbm.at[srcb.at[b]], sbuf[b], gss[b])

    def outer(gi, carry):
        for b in range(GNBUF):
            j = gi * GNBUF + b
            ebase = wid * EPW + j * CHUNK
            pltpu.make_async_copy(dtab_hbm.at[dstb.at[b]], dbuf[b], gsd[b]).wait()
            pltpu.make_async_copy(stab_hbm.at[srcb.at[b]], sbuf[b], gss[b]).wait()
            cw1 = pltpu.async_copy(dbuf[b], dg_hbm.at[pl.ds(ebase, CHUNK)], wsd[b])
            cw2 = pltpu.async_copy(sbuf[b], sg_hbm.at[pl.ds(ebase, CHUNK)], wss[b])
            cw1.wait()
            cw2.wait()
            nj = j + GNBUF

            @pl.when(nj < NCHUNKS)
            def _():
                pltpu.async_copy(dtab_hbm.at[dstb.at[nj]], dbuf[b], gsd[b])
                pltpu.async_copy(stab_hbm.at[srcb.at[nj]], sbuf[b], gss[b])
        return carry

    lax.fori_loop(0, NCHUNKS // GNBUF, outer, 0)


def _gather(dtab, stab, dst2, src2, SW):
    mesh = plsc.VectorSubcoreMesh(core_axis_name="c", subcore_axis_name="s")
    kfn = pl.kernel(
        functools.partial(_gather_body, SW=SW),
        out_type=[jax.ShapeDtypeStruct((EPAD, 128), _f32),
                  jax.ShapeDtypeStruct((EPAD, SW), _f32)],
        mesh=mesh,
        scratch_types=(
            [pltpu.VMEM((NCHUNKS, CHUNK), jnp.int32),
             pltpu.VMEM((NCHUNKS, CHUNK), jnp.int32)]
            + [pltpu.VMEM((CHUNK, 128), _f32)] * GNBUF
            + [pltpu.VMEM((CHUNK, SW), _f32)] * GNBUF
            + [pltpu.SemaphoreType.DMA] * (4 * GNBUF)
        ))
    return kfn(dtab, stab, dst2, src2)


SNBUF = 2


def _scatter_body(msg_hbm, dst2, zrow_hbm, accs_hbm, *scr):
    dstb, acc_sh = scr[0], scr[1]
    msgv = scr[2:2 + SNBUF]
    rsem = scr[2 + SNBUF:2 + 2 * SNBUF]
    asem = scr[2 + 2 * SNBUF:2 + 3 * SNBUF]
    zsem = scr[2 + 3 * SNBUF]
    c = lax.axis_index("c")
    s = lax.axis_index("s")
    wid = s * NC + c
    pltpu.sync_copy(dst2.at[pl.ds(wid * NCHUNKS, NCHUNKS)], dstb)
    zv = msgv[0]
    pltpu.sync_copy(zrow_hbm, zv)
    zcs = [pltpu.async_copy(
        zv, acc_sh.at[pl.ds(s * ROWS_PER_TILE + r * CHUNK, CHUNK)], zsem)
        for r in range(ROWS_PER_TILE // CHUNK)]
    for cp in zcs:
        cp.wait()
    plsc.subcore_barrier()

    for b in range(SNBUF):
        ebase = wid * EPW + b * CHUNK
        pltpu.async_copy(msg_hbm.at[pl.ds(ebase, CHUNK)], msgv[b], rsem[b])

    def outer(gi, carry):
        for b in range(SNBUF):
            j = gi * SNBUF + b
            ebase = wid * EPW + j * CHUNK
            pltpu.make_async_copy(
                msg_hbm.at[pl.ds(ebase, CHUNK)], msgv[b], rsem[b]).wait()
            ca = pltpu.async_copy(msgv[b], acc_sh.at[dstb.at[j]], asem[b],
                                  add=True)
            ca.wait()
            nj = j + SNBUF

            @pl.when(nj < NCHUNKS)
            def _():
                nbase = wid * EPW + nj * CHUNK
                pltpu.async_copy(msg_hbm.at[pl.ds(nbase, CHUNK)], msgv[b],
                                 rsem[b])
        return carry

    lax.fori_loop(0, NCHUNKS // SNBUF, outer, 0)
    plsc.subcore_barrier()

    nwb = ROWS_PER_TILE // CHUNK
    pltpu.async_copy(acc_sh.at[pl.ds(s * ROWS_PER_TILE, CHUNK)], msgv[0],
                     rsem[0])
    for r in range(nwb):
        rows = s * ROWS_PER_TILE + r * CHUNK
        pltpu.make_async_copy(
            acc_sh.at[pl.ds(rows, CHUNK)], msgv[r % 2], rsem[r % 2]).wait()
        if r + 1 < nwb:
            nrows = rows + CHUNK
            pltpu.async_copy(acc_sh.at[pl.ds(nrows, CHUNK)], msgv[(r + 1) % 2],
                             rsem[(r + 1) % 2])
        pltpu.sync_copy(msgv[r % 2], accs_hbm.at[c, pl.ds(rows, CHUNK)])


def _scatter(msg, dst2):
    mesh = plsc.VectorSubcoreMesh(core_axis_name="c", subcore_axis_name="s")
    zrow = jnp.zeros((CHUNK, AW), _f32)
    kfn = pl.kernel(
        _scatter_body,
        out_type=jax.ShapeDtypeStruct((NC, NPAD, AW), _f32),
        mesh=mesh,
        scratch_types=(
            [pltpu.VMEM((NCHUNKS, CHUNK), jnp.int32),
             pltpu.VMEM_SHARED((NPAD, AW), _f32)]
            + [pltpu.VMEM((CHUNK, AW), _f32)] * SNBUF
            + [pltpu.SemaphoreType.DMA] * (2 * SNBUF + 1)
        ))
    return kfn(msg, dst2, zrow)


def kernel(x, edge_index, edge_attr, train_masked_y, eval_masked_y, params):
    del train_masked_y
    src = edge_index[0]
    dst = edge_index[1]
    h = jnp.zeros((NPAD, IN_DIM), _f32).at[:N].set(x)
    y = jnp.zeros((NPAD, LABEL_DIM), _f32).at[:N].set(eval_masked_y)
    ea = jnp.concatenate(
        [edge_attr, jnp.ones((E, 1), _f32), jnp.zeros((E, 7), _f32)], axis=1)
    ea = jnp.zeros((EPAD, 16), _f32).at[:E].set(ea)
    src2 = jnp.zeros((EPAD,), jnp.int32).at[:E].set(src).reshape(EPAD // CHUNK, CHUNK)
    dst2 = jnp.full((EPAD,), DUMP, jnp.int32).at[:E].set(dst).reshape(EPAD // CHUNK, CHUNK)

    layers = params["layers"]
    for i, (C, relu) in enumerate([(64, True), (64, True), (112, False)]):
        SW = 128 if 2 * C <= 128 else 256
        p = layers[i]
        dtab, stab, skip = _proj(h, y, p, C, SW)
        dg, sg = _gather(dtab, stab, dst2, src2, SW)
        l, m = _logits(dg, sg, ea, C, SW)
        msg = _msg(l, m, sg, C, SW)
        accs = _scatter(msg, dst2)
        h = _combine(accs[0], accs[1], skip, C, relu)
    return h[:N]


# speedup vs baseline: 5.7744x; 1.0001x over previous
import functools
import math

import jax
import jax.numpy as jnp
from jax import lax
from jax.experimental import pallas as pl
from jax.experimental.pallas import tpu as pltpu
from jax.experimental.pallas import tpu_sc as plsc

N = 10000
E = 160000
IN_DIM = 8
LABEL_DIM = 112

NPAD = 10240
DUMP = N
EPAD = 163840
NC, NS = 2, 16
NW = NC * NS
EPW = EPAD // NW
CHUNK = 128
NCHUNKS = EPW // CHUNK
ROWS_PER_TILE = NPAD // NS
NODE_BLK = 1024
EDGE_BLK = 2048
AW = 128

_f32 = jnp.float32


def _proj_body(h_ref, y_ref, wq, bq, wkn, bkn, wv, bv, wl, bl, ws, bs,
               wket, bkec, dtab_ref, stab_ref, skip_ref, *, C, SW):
    h = h_ref[...]
    y = y_ref[...]
    b = h.shape[0]
    q = jnp.dot(h, wq[...], preferred_element_type=_f32) + bq[...]
    kn = jnp.dot(h, wkn[...], preferred_element_type=_f32) + bkn[...]
    v = jnp.dot(h, wv[...], preferred_element_type=_f32) + bv[...]
    labp = jnp.dot(y, wl[...], preferred_element_type=_f32) + bl[...]
    w = v + labp
    skip_ref[...] = jnp.dot(h, ws[...], preferred_element_type=_f32) + bs[...]
    u = jnp.dot(q, wket[...], preferred_element_type=_f32)
    cterm = jnp.dot(q, bkec[...], preferred_element_type=_f32)
    dparts = [q, u, cterm]
    if 128 - C - 9 > 0:
        dparts.append(jnp.zeros((b, 128 - C - 9), _f32))
    dtab_ref[...] = jnp.concatenate(dparts, axis=1)
    sparts = [kn, w]
    if SW - 2 * C > 0:
        sparts.append(jnp.zeros((b, SW - 2 * C), _f32))
    stab_ref[...] = jnp.concatenate(sparts, axis=1)


def _proj(h, y, p, C, SW):
    din = h.shape[1]
    grid = NPAD // NODE_BLK
    row_spec = lambda width: pl.BlockSpec((NODE_BLK, width), lambda i: (i, 0))
    full = lambda a: pl.BlockSpec(a.shape, lambda i: (0,) * a.ndim)
    wq, bq = p["Wq"], p["bq"].reshape(1, C)
    wkn, bkn = p["Wkn"], p["bkn"].reshape(1, C)
    wv, bv = p["Wv"], p["bv"].reshape(1, C)
    wl, bl = p["Wl"], p["bl"].reshape(1, C)
    ws, bs = p["Ws"], p["bs"].reshape(1, C)
    wket = p["Wke"].T
    bkec = p["bke"].reshape(C, 1)
    args = (h, y, wq, bq, wkn, bkn, wv, bv, wl, bl, ws, bs, wket, bkec)
    in_specs = [row_spec(din), row_spec(LABEL_DIM)] + [full(a) for a in args[2:]]
    out_sd = [jax.ShapeDtypeStruct((NPAD, 128), _f32),
              jax.ShapeDtypeStruct((NPAD, SW), _f32),
              jax.ShapeDtypeStruct((NPAD, C), _f32)]
    out_specs = [row_spec(128), row_spec(SW), row_spec(C)]
    return pl.pallas_call(
        functools.partial(_proj_body, C=C, SW=SW),
        grid=(grid,), in_specs=in_specs, out_specs=out_specs,
        out_shape=out_sd)(*args)


def _logits_body(dg, sg, ea, l_ref, m_ref, mmax, *, C, inv_sqrt_c):
    i = pl.program_id(0)
    d = dg[...]
    qg = d[:, :C]
    ubg = d[:, C:C + 16]
    kng = sg[...][:, :C]
    dots = (jnp.sum(qg * kng, axis=1, keepdims=True)
            + jnp.sum(ubg * ea[...], axis=1, keepdims=True))
    l = dots * inv_sqrt_c
    l_ref[...] = l
    bm = jnp.max(l)

    @pl.when(i == 0)
    def _():
        mmax[0] = bm

    @pl.when(i > 0)
    def _():
        mmax[0] = jnp.maximum(mmax[0], bm)

    @pl.when(i == pl.num_programs(0) - 1)
    def _():
        m_ref[...] = jnp.broadcast_to(mmax[0], (1, 1))


def _logits(dg, sg, ea, C, SW):
    grid = EPAD // EDGE_BLK
    spec = lambda width: pl.BlockSpec((EDGE_BLK, width), lambda i: (i, 0))
    return pl.pallas_call(
        functools.partial(_logits_body, C=C, inv_sqrt_c=1.0 / math.sqrt(C)),
        grid=(grid,),
        in_specs=[spec(128), spec(SW), spec(16)],
        out_specs=[spec(1), pl.BlockSpec((1, 1), lambda i: (0, 0))],
        out_shape=[jax.ShapeDtypeStruct((EPAD, 1), _f32),
                   jax.ShapeDtypeStruct((1, 1), _f32)],
        scratch_shapes=[pltpu.SMEM((1,), _f32)])(dg, sg, ea)


def _msg_body(l_ref, m_ref, sg_ref, msg_ref, *, C):
    p = jnp.exp(l_ref[...] - m_ref[0, 0])
    wg = sg_ref[...][:, C:2 * C]
    msg_ref[...] = jnp.concatenate(
        [wg * p, p, jnp.zeros((p.shape[0], AW - C - 1), _f32)], axis=1)


def _msg(l, m, sg, C, SW):
    grid = EPAD // EDGE_BLK
    spec = lambda width: pl.BlockSpec((EDGE_BLK, width), lambda i: (i, 0))
    return pl.pallas_call(
        functools.partial(_msg_body, C=C), grid=(grid,),
        in_specs=[spec(1), pl.BlockSpec((1, 1), lambda i: (0, 0)), spec(SW)],
        out_specs=spec(AW),
        out_shape=jax.ShapeDtypeStruct((EPAD, AW), _f32))(l, m, sg)


def _combine_body(a0_ref, a1_ref, skip_ref, out_ref, *, C, relu):
    a = a0_ref[...] + a1_ref[...]
    o = a[:, :C] / (a[:, C:C + 1] + 1e-16) + skip_ref[...]
    if relu:
        o = jnp.maximum(o, 0.0)
    out_ref[...] = o


def _combine(acc0, acc1, skip, C, relu):
    grid = NPAD // NODE_BLK
    spec = lambda width: pl.BlockSpec((NODE_BLK, width), lambda i: (i, 0))
    return pl.pallas_call(
        functools.partial(_combine_body, C=C, relu=relu), grid=(grid,),
        in_specs=[spec(AW), spec(AW), spec(C)],
        out_specs=spec(C),
        out_shape=jax.ShapeDtypeStruct((NPAD, C), _f32))(acc0, acc1, skip)


def _gather_body(dtab_hbm, stab_hbm, dst2, src2, dg_hbm, sg_hbm, *scr, SW, GNBUF):
    dstb, srcb = scr[0], scr[1]
    dbuf = scr[2:2 + GNBUF]
    sbuf = scr[2 + GNBUF:2 + 2 * GNBUF]
    gsd = scr[2 + 2 * GNBUF:2 + 3 * GNBUF]
    gss = scr[2 + 3 * GNBUF:2 + 4 * GNBUF]
    wsd = scr[2 + 4 * GNBUF:2 + 5 * GNBUF]
    wss = scr[2 + 5 * GNBUF:2 + 6 * GNBUF]
    c = lax.axis_index("c")
    s = lax.axis_index("s")
    wid = s * NC + c
    pltpu.sync_copy(dst2.at[pl.ds(wid * NCHUNKS, NCHUNKS)], dstb)
    pltpu.sync_copy(src2.at[pl.ds(wid * NCHUNKS, NCHUNKS)], srcb)

    for b in range(GNBUF):
        pltpu.async_copy(dtab_hbm.at[dstb.at[b]], dbuf[b], gsd[b])
        pltpu.async_copy(stab_hbm.at[srcb.at[b]], sbuf[b], gss[b])

    def outer(gi, carry):
        for b in range(GNBUF):
            j = gi * GNBUF + b
            ebase = wid * EPW + j * CHUNK
            pltpu.make_async_copy(dtab_hbm.at[dstb.at[b]], dbuf[b], gsd[b]).wait()
            pltpu.make_async_copy(stab_hbm.at[srcb.at[b]], sbuf[b], gss[b]).wait()
            cw1 = pltpu.async_copy(dbuf[b], dg_hbm.at[pl.ds(ebase, CHUNK)], wsd[b])
            cw2 = pltpu.async_copy(sbuf[b], sg_hbm.at[pl.ds(ebase, CHUNK)], wss[b])
            cw1.wait()
            cw2.wait()
            nj = j + GNBUF

            @pl.when(nj < NCHUNKS)
            def _():
                pltpu.async_copy(dtab_hbm.at[dstb.at[nj]], dbuf[b], gsd[b])
                pltpu.async_copy(stab_hbm.at[srcb.at[nj]], sbuf[b], gss[b])
        return carry

    lax.fori_loop(0, NCHUNKS // GNBUF, outer, 0)

    for j in range((NCHUNKS // GNBUF) * GNBUF, NCHUNKS):
        b = j % GNBUF
        ebase = wid * EPW + j * CHUNK
        pltpu.make_async_copy(dtab_hbm.at[dstb.at[b]], dbuf[b], gsd[b]).wait()
        pltpu.make_async_copy(stab_hbm.at[srcb.at[b]], sbuf[b], gss[b]).wait()
        cw1 = pltpu.async_copy(dbuf[b], dg_hbm.at[pl.ds(ebase, CHUNK)], wsd[b])
        cw2 = pltpu.async_copy(sbuf[b], sg_hbm.at[pl.ds(ebase, CHUNK)], wss[b])
        cw1.wait()
        cw2.wait()


def _gather(dtab, stab, dst2, src2, SW):
    GNBUF = 2 if SW > 128 else 3
    mesh = plsc.VectorSubcoreMesh(core_axis_name="c", subcore_axis_name="s")
    kfn = pl.kernel(
        functools.partial(_gather_body, SW=SW, GNBUF=GNBUF),
        out_type=[jax.ShapeDtypeStruct((EPAD, 128), _f32),
                  jax.ShapeDtypeStruct((EPAD, SW), _f32)],
        mesh=mesh,
        scratch_types=(
            [pltpu.VMEM((NCHUNKS, CHUNK), jnp.int32),
             pltpu.VMEM((NCHUNKS, CHUNK), jnp.int32)]
            + [pltpu.VMEM((CHUNK, 128), _f32)] * GNBUF
            + [pltpu.VMEM((CHUNK, SW), _f32)] * GNBUF
            + [pltpu.SemaphoreType.DMA] * (4 * GNBUF)
        ))
    return kfn(dtab, stab, dst2, src2)


SNBUF = 2


def _scatter_body(msg_hbm, dst2, zrow_hbm, accs_hbm, *scr):
    dstb, acc_sh = scr[0], scr[1]
    msgv = scr[2:2 + SNBUF]
    rsem = scr[2 + SNBUF:2 + 2 * SNBUF]
    asem = scr[2 + 2 * SNBUF:2 + 3 * SNBUF]
    zsem = scr[2 + 3 * SNBUF]
    c = lax.axis_index("c")
    s = lax.axis_index("s")
    wid = s * NC + c
    pltpu.sync_copy(dst2.at[pl.ds(wid * NCHUNKS, NCHUNKS)], dstb)
    zv = msgv[0]
    pltpu.sync_copy(zrow_hbm, zv)
    zcs = [pltpu.async_copy(
        zv, acc_sh.at[pl.ds(s * ROWS_PER_TILE + r * CHUNK, CHUNK)], zsem)
        for r in range(ROWS_PER_TILE // CHUNK)]
    for cp in zcs:
        cp.wait()
    plsc.subcore_barrier()

    for b in range(SNBUF):
        ebase = wid * EPW + b * CHUNK
        pltpu.async_copy(msg_hbm.at[pl.ds(ebase, CHUNK)], msgv[b], rsem[b])

    def outer(gi, carry):
        for b in range(SNBUF):
            j = gi * SNBUF + b
            ebase = wid * EPW + j * CHUNK
            pltpu.make_async_copy(
                msg_hbm.at[pl.ds(ebase, CHUNK)], msgv[b], rsem[b]).wait()
            ca = pltpu.async_copy(msgv[b], acc_sh.at[dstb.at[j]], asem[b],
                                  add=True)
            ca.wait()
            nj = j + SNBUF

            @pl.when(nj < NCHUNKS)
            def _():
                nbase = wid * EPW + nj * CHUNK
                pltpu.async_copy(msg_hbm.at[pl.ds(nbase, CHUNK)], msgv[b],
                                 rsem[b])
        return carry

    lax.fori_loop(0, NCHUNKS // SNBUF, outer, 0)
    plsc.subcore_barrier()

    nwb = ROWS_PER_TILE // CHUNK
    pltpu.async_copy(acc_sh.at[pl.ds(s * ROWS_PER_TILE, CHUNK)], msgv[0],
                     rsem[0])
    for r in range(nwb):
        rows = s * ROWS_PER_TILE + r * CHUNK
        pltpu.make_async_copy(
            acc_sh.at[pl.ds(rows, CHUNK)], msgv[r % 2], rsem[r % 2]).wait()
        if r + 1 < nwb:
            nrows = rows + CHUNK
            pltpu.async_copy(acc_sh.at[pl.ds(nrows, CHUNK)], msgv[(r + 1) % 2],
                             rsem[(r + 1) % 2])
        pltpu.sync_copy(msgv[r % 2], accs_hbm.at[c, pl.ds(rows, CHUNK)])


def _scatter(msg, dst2):
    mesh = plsc.VectorSubcoreMesh(core_axis_name="c", subcore_axis_name="s")
    zrow = jnp.zeros((CHUNK, AW), _f32)
    kfn = pl.kernel(
        _scatter_body,
        out_type=jax.ShapeDtypeStruct((NC, NPAD, AW), _f32),
        mesh=mesh,
        scratch_types=(
            [pltpu.VMEM((NCHUNKS, CHUNK), jnp.int32),
             pltpu.VMEM_SHARED((NPAD, AW), _f32)]
            + [pltpu.VMEM((CHUNK, AW), _f32)] * SNBUF
            + [pltpu.SemaphoreType.DMA] * (2 * SNBUF + 1)
        ))
    return kfn(msg, dst2, zrow)


def kernel(x, edge_index, edge_attr, train_masked_y, eval_masked_y, params):
    del train_masked_y
    src = edge_index[0]
    dst = edge_index[1]
    h = jnp.zeros((NPAD, IN_DIM), _f32).at[:N].set(x)
    y = jnp.zeros((NPAD, LABEL_DIM), _f32).at[:N].set(eval_masked_y)
    ea = jnp.concatenate(
        [edge_attr, jnp.ones((E, 1), _f32), jnp.zeros((E, 7), _f32)], axis=1)
    ea = jnp.zeros((EPAD, 16), _f32).at[:E].set(ea)
    src2 = jnp.zeros((EPAD,), jnp.int32).at[:E].set(src).reshape(EPAD // CHUNK, CHUNK)
    dst2 = jnp.full((EPAD,), DUMP, jnp.int32).at[:E].set(dst).reshape(EPAD // CHUNK, CHUNK)

    layers = params["layers"]
    for i, (C, relu) in enumerate([(64, True), (64, True), (112, False)]):
        SW = 128 if 2 * C <= 128 else 256
        p = layers[i]
        dtab, stab, skip = _proj(h, y, p, C, SW)
        dg, sg = _gather(dtab, stab, dst2, src2, SW)
        l, m = _logits(dg, sg, ea, C, SW)
        msg = _msg(l, m, sg, C, SW)
        accs = _scatter(msg, dst2)
        h = _combine(accs[0], accs[1], skip, C, relu)
    return h[:N]
